# baseline (speedup 1.0000x reference)
import functools

import jax
import jax.numpy as jnp
from jax import lax
from jax.experimental import pallas as pl
from jax.experimental.pallas import tpu as pltpu
from jax.experimental.pallas import tpu_sc as plsc

N = 10000
D = 128
DE = 16
NC, NS = 2, 16
NW = NC * NS
CH = 128
RPW = 640
R = NS * RPW
LANES = 16


def _lin_body(x_ref, w_ref, b_ref, o_ref):
    o_ref[...] = (
        jnp.dot(x_ref[...], w_ref[...], preferred_element_type=jnp.float32)
        + b_ref[...]
    )


def _node_linear(x, w, b):
    blk = 2000
    return pl.pallas_call(
        _lin_body,
        grid=(N // blk,),
        in_specs=[
            pl.BlockSpec((blk, D), lambda i: (i, 0)),
            pl.BlockSpec((D, D), lambda i: (0, 0)),
            pl.BlockSpec((D,), lambda i: (0,)),
        ],
        out_specs=pl.BlockSpec((blk, D), lambda i: (i, 0)),
        out_shape=jax.ShapeDtypeStruct((N, D), jnp.float32),
    )(x, w, b)


CH2 = 2048


def _sc_fused_body(xl_hbm, src_hbm, eat_hbm, dst_hbm, z_big,
                   agg_hbm, sat_hbm,
                   src0, src1, dst_v, rows0, rows1, dstc_v, valc_v, acc2_v,
                   sem0, sem1, acc, n0, n1):
    c = lax.axis_index("c")
    s = lax.axis_index("s")
    row0 = s * RPW
    n_sub = RPW // CH
    nc = jnp.where(c == 0, n0, n1)
    eb = c * (16 * n0) * CH

    pltpu.sync_copy(z_big, rows0)
    for r in range(n_sub):
        pltpu.sync_copy(rows0, acc.at[pl.ds(row0 + r * CH, CH)])

    def zero(i, carry):
        acc2_v[pl.ds(i * LANES, LANES)] = jnp.zeros((LANES,), jnp.float32)
        return carry

    lax.fori_loop(0, R // LANES, zero, 0)
    plsc.subcore_barrier()

    base = eb + s * nc * CH
    ibufs = (src0, src1)
    bufs = (rows0, rows1)
    sems = (sem0, sem1)

    def gather(j, ibuf, buf, sem):
        pltpu.sync_copy(src_hbm.at[pl.ds(base + j * CH, CH)], ibuf)
        pltpu.async_copy(xl_hbm.at[ibuf], buf, sem)

    def ea_step(j, carry):
        off = eb + j * CH2
        pltpu.sync_copy(dst_hbm.at[pl.ds(off, CH2)], dstc_v)
        pltpu.sync_copy(eat_hbm.at[s, pl.ds(off, CH2)], valc_v)
        for i in range(CH2 // LANES):
            idx = dstc_v[pl.ds(i * LANES, LANES)]
            val = valc_v[pl.ds(i * LANES, LANES)]
            plsc.addupdate_scatter(acc2_v, [idx], val)
        return carry

    gather(0, src0, rows0, sem0)

    def outer(g, carry):
        for b in range(2):
            j = g * 2 + b

            @pl.when(j + 1 < nc)
            def _():
                gather(j + 1, ibufs[1 - b], bufs[1 - b], sems[1 - b])

            ea_step(j, 0)
            pltpu.make_async_copy(xl_hbm.at[ibufs[b]], bufs[b], sems[b]).wait()
            pltpu.sync_copy(dst_hbm.at[pl.ds(base + j * CH, CH)], dst_v)
            pltpu.sync_copy(bufs[b], acc.at[dst_v], add=True)
        return carry

    lax.fori_loop(0, nc // 2, outer, 0)
    plsc.subcore_barrier()

    out0 = c * R + row0
    for r in range(n_sub):
        pltpu.sync_copy(acc.at[pl.ds(row0 + r * CH, CH)], rows0)
        pltpu.sync_copy(rows0, agg_hbm.at[pl.ds(out0 + r * CH, CH)])
    pltpu.sync_copy(acc2_v, sat_hbm.at[pl.ds((c * DE + s) * R, R)])


def _sc_fused(xl, src_p, dst_p, ea_t, n0, n1):
    mesh = plsc.VectorSubcoreMesh(core_axis_name="c", subcore_axis_name="s")
    kfn = functools.partial(
        pl.kernel,
        out_type=[
            jax.ShapeDtypeStruct((NC * R, D), jnp.float32),
            jax.ShapeDtypeStruct((NC * DE * R,), jnp.float32),
        ],
        mesh=mesh,
        compiler_params=pltpu.CompilerParams(needs_layout_passes=False),
        scratch_types=[
            pltpu.VMEM((CH,), jnp.int32),
            pltpu.VMEM((CH,), jnp.int32),
            pltpu.VMEM((CH,), jnp.int32),
            pltpu.VMEM((CH, D), jnp.float32),
            pltpu.VMEM((CH, D), jnp.float32),
            pltpu.VMEM((CH2,), jnp.int32),
            pltpu.VMEM((CH2,), jnp.float32),
            pltpu.VMEM((R,), jnp.float32),
            pltpu.SemaphoreType.DMA,
            pltpu.SemaphoreType.DMA,
            pltpu.VMEM_SHARED((R, D), jnp.float32),
        ],
    )(functools.partial(_sc_fused_body, n0=n0, n1=n1))
    return kfn(xl, src_p, ea_t, dst_p, jnp.zeros((CH, D), jnp.float32))


def _comb_body(p_ref, t_ref, xl_ref, w_ref, o_ref):
    svec = t_ref[0] + t_ref[1]
    o_ref[...] = (
        p_ref[0]
        + p_ref[1]
        + xl_ref[...]
        + jnp.dot(svec, w_ref[...], preferred_element_type=jnp.float32)
        + jnp.sum(w_ref[...], axis=0)[None, :]
    )


def _combine(agg, sat, xl, w_ext):
    blk = 2000
    return pl.pallas_call(
        _comb_body,
        grid=(N // blk,),
        in_specs=[
            pl.BlockSpec((NC, blk, D), lambda i: (0, i, 0)),
            pl.BlockSpec((NC, blk, DE), lambda i: (0, i, 0)),
            pl.BlockSpec((blk, D), lambda i: (i, 0)),
            pl.BlockSpec((DE, D), lambda i: (0, 0)),
        ],
        out_specs=pl.BlockSpec((blk, D), lambda i: (i, 0)),
        out_shape=jax.ShapeDtypeStruct((N, D), jnp.float32),
    )(agg, sat, xl, w_ext)


def kernel(x, edge_index, edge_attr, W_lin, b_lin, W_edge, b_edge):
    E = edge_index.shape[1]
    n_chunks = -(-E // (NW * CH))
    n_chunks += n_chunks % 2
    EP = NW * CH * n_chunks
    tot16 = EP // (CH * 16)
    n0 = max(2, (int(tot16 * 0.375) // 2) * 2)
    n1 = tot16 - n0
    src = edge_index[0].astype(jnp.int32)
    dst = edge_index[1].astype(jnp.int32)
    pad = EP - E
    src_p = jnp.concatenate([src, jnp.zeros((pad,), jnp.int32)])
    dst_p = jnp.concatenate([dst, jnp.full((pad,), N, jnp.int32)])
    ea_ext = jnp.concatenate(
        [edge_attr,
         jnp.ones((E, 1), jnp.float32),
         jnp.zeros((E, DE - edge_attr.shape[1] - 1), jnp.float32)], axis=1)
    ea_t = jnp.concatenate(
        [ea_ext, jnp.zeros((pad, DE), jnp.float32)], axis=0).T
    w_ext = jnp.concatenate(
        [W_edge, b_edge[None, :],
         jnp.zeros((DE - W_edge.shape[0] - 1, D), jnp.float32)], axis=0)

    xl = _node_linear(x, W_lin, b_lin)
    agg, sat = _sc_fused(xl, src_p, dst_p, ea_t, n0, n1)
    agg = agg.reshape(NC, R, D)[:, :N, :]
    sat = sat.reshape(NC, DE, R)[:, :, :N].transpose(0, 2, 1)
    return _combine(agg, sat, xl, w_ext)


# speedup vs baseline: 1.2525x; 1.2525x over previous
import functools

import jax
import jax.numpy as jnp
from jax import lax
from jax.experimental import pallas as pl
from jax.experimental.pallas import tpu as pltpu
from jax.experimental.pallas import tpu_sc as plsc

N = 10000
D = 128
DE = 16
NC, NS = 2, 16
NW = NC * NS
CH = 128
RPW = 640
R = NS * RPW
LANES = 16


def _lin_body(x_ref, w_ref, b_ref, o_ref):
    o_ref[...] = (
        jnp.dot(x_ref[...], w_ref[...], preferred_element_type=jnp.float32)
        + b_ref[...]
    )


def _node_linear(x, w, b):
    blk = 2000
    return pl.pallas_call(
        _lin_body,
        grid=(N // blk,),
        in_specs=[
            pl.BlockSpec((blk, D), lambda i: (i, 0)),
            pl.BlockSpec((D, D), lambda i: (0, 0)),
            pl.BlockSpec((D,), lambda i: (0,)),
        ],
        out_specs=pl.BlockSpec((blk, D), lambda i: (i, 0)),
        out_shape=jax.ShapeDtypeStruct((N, D), jnp.float32),
    )(x, w, b)


CH2 = 2048


def _sc_fused_body(xl_hbm, src_hbm, eat_hbm, dst_hbm, z_big,
                   agg_hbm, sat_hbm,
                   src0, src1, dst_v, rows0, rows1, dstc_v, valc_v, acc2_v,
                   sem0, sem1, acc, n0, n1):
    c = lax.axis_index("c")
    s = lax.axis_index("s")
    row0 = s * RPW
    n_sub = RPW // CH
    nc = jnp.where(c == 0, n0, n1)
    eb = c * (16 * n0) * CH

    pltpu.sync_copy(z_big, rows0)
    for r in range(n_sub):
        pltpu.sync_copy(rows0, acc.at[pl.ds(row0 + r * CH, CH)])

    def zero(i, carry):
        acc2_v[pl.ds(i * LANES, LANES)] = jnp.zeros((LANES,), jnp.float32)
        return carry

    lax.fori_loop(0, R // LANES, zero, 0)
    plsc.subcore_barrier()

    base = eb + s * nc * CH
    ibufs = (src0, src1)
    bufs = (rows0, rows1)
    sems = (sem0, sem1)

    def gather(j, ibuf, buf, sem):
        pltpu.sync_copy(src_hbm.at[pl.ds(base + j * CH, CH)], ibuf)
        pltpu.async_copy(xl_hbm.at[ibuf], buf, sem)

    def ea_step(j, carry):
        off = eb + j * CH2
        pltpu.sync_copy(dst_hbm.at[pl.ds(off, CH2)], dstc_v)
        pltpu.sync_copy(eat_hbm.at[s, pl.ds(off, CH2)], valc_v)
        for i in range(CH2 // LANES):
            idx = dstc_v[pl.ds(i * LANES, LANES)]
            val = valc_v[pl.ds(i * LANES, LANES)]
            plsc.addupdate_scatter(acc2_v, [idx], val)
        return carry

    gather(0, src0, rows0, sem0)

    def outer(g, carry):
        for b in range(2):
            j = g * 2 + b

            @pl.when(j + 1 < nc)
            def _():
                gather(j + 1, ibufs[1 - b], bufs[1 - b], sems[1 - b])

            ea_step(j, 0)
            pltpu.make_async_copy(xl_hbm.at[ibufs[b]], bufs[b], sems[b]).wait()
            pltpu.sync_copy(dst_hbm.at[pl.ds(base + j * CH, CH)], dst_v)
            pltpu.sync_copy(bufs[b], acc.at[dst_v], add=True)
        return carry

    lax.fori_loop(0, nc // 2, outer, 0)
    plsc.subcore_barrier()

    out0 = c * R + row0
    for r in range(n_sub):
        pltpu.sync_copy(acc.at[pl.ds(row0 + r * CH, CH)], rows0)
        pltpu.sync_copy(rows0, agg_hbm.at[pl.ds(out0 + r * CH, CH)])
    pltpu.sync_copy(acc2_v, sat_hbm.at[pl.ds((c * DE + s) * R, R)])


def _sc_fused(xl, src_p, dst_p, ea_t, n0, n1):
    mesh = plsc.VectorSubcoreMesh(core_axis_name="c", subcore_axis_name="s")
    kfn = functools.partial(
        pl.kernel,
        out_type=[
            jax.ShapeDtypeStruct((NC * R, D), jnp.float32),
            jax.ShapeDtypeStruct((NC * DE * R,), jnp.float32),
        ],
        mesh=mesh,
        compiler_params=pltpu.CompilerParams(needs_layout_passes=False),
        scratch_types=[
            pltpu.VMEM((CH,), jnp.int32),
            pltpu.VMEM((CH,), jnp.int32),
            pltpu.VMEM((CH,), jnp.int32),
            pltpu.VMEM((CH, D), jnp.float32),
            pltpu.VMEM((CH, D), jnp.float32),
            pltpu.VMEM((CH2,), jnp.int32),
            pltpu.VMEM((CH2,), jnp.float32),
            pltpu.VMEM((R,), jnp.float32),
            pltpu.SemaphoreType.DMA,
            pltpu.SemaphoreType.DMA,
            pltpu.VMEM_SHARED((R, D), jnp.float32),
        ],
    )(functools.partial(_sc_fused_body, n0=n0, n1=n1))
    return kfn(xl, src_p, ea_t, dst_p, jnp.zeros((CH, D), jnp.float32))


def _comb_body(p_ref, t_ref, xl_ref, w_ref, o_ref):
    svec = t_ref[0] + t_ref[1]
    o_ref[...] = (
        p_ref[0]
        + p_ref[1]
        + xl_ref[...]
        + jnp.dot(svec, w_ref[...], preferred_element_type=jnp.float32)
        + jnp.sum(w_ref[...], axis=0)[None, :]
    )


def _combine(agg, sat, xl, w_ext):
    blk = 2000
    return pl.pallas_call(
        _comb_body,
        grid=(N // blk,),
        in_specs=[
            pl.BlockSpec((NC, blk, D), lambda i: (0, i, 0)),
            pl.BlockSpec((NC, blk, DE), lambda i: (0, i, 0)),
            pl.BlockSpec((blk, D), lambda i: (i, 0)),
            pl.BlockSpec((DE, D), lambda i: (0, 0)),
        ],
        out_specs=pl.BlockSpec((blk, D), lambda i: (i, 0)),
        out_shape=jax.ShapeDtypeStruct((N, D), jnp.float32),
    )(agg, sat, xl, w_ext)


def kernel(x, edge_index, edge_attr, W_lin, b_lin, W_edge, b_edge):
    E = edge_index.shape[1]
    n_chunks = -(-E // (NW * CH))
    n_chunks += n_chunks % 2
    EP = NW * CH * n_chunks
    tot16 = EP // (CH * 16)
    n1 = max(2, (int(tot16 * 0.375) // 2) * 2)
    n0 = tot16 - n1
    src = edge_index[0].astype(jnp.int32)
    dst = edge_index[1].astype(jnp.int32)
    pad = EP - E
    src_p = jnp.concatenate([src, jnp.zeros((pad,), jnp.int32)])
    dst_p = jnp.concatenate([dst, jnp.full((pad,), N, jnp.int32)])
    ea_ext = jnp.concatenate(
        [edge_attr,
         jnp.ones((E, 1), jnp.float32),
         jnp.zeros((E, DE - edge_attr.shape[1] - 1), jnp.float32)], axis=1)
    ea_t = jnp.concatenate(
        [ea_ext, jnp.zeros((pad, DE), jnp.float32)], axis=0).T
    w_ext = jnp.concatenate(
        [W_edge, b_edge[None, :],
         jnp.zeros((DE - W_edge.shape[0] - 1, D), jnp.float32)], axis=0)

    xl = _node_linear(x, W_lin, b_lin)
    agg, sat = _sc_fused(xl, src_p, dst_p, ea_t, n0, n1)
    agg = agg.reshape(NC, R, D)[:, :N, :]
    sat = sat.reshape(NC, DE, R)[:, :, :N].transpose(0, 2, 1)
    return _combine(agg, sat, xl, w_ext)
